# Initial kernel scaffold; baseline (speedup 1.0000x reference)
import jax
import jax.numpy as jnp
from jax.experimental import pallas as pl


def kernel(x, ei, ew, Wz, bz, Lz_w, Lz_b, Wr, br, Lr_w, Lr_b, Wh, bh, Lh_w, Lh_b, head_w, head_b):
    raise NotImplementedError("write your pallas kernel here")


# speedup vs baseline: 24.6006x; 24.6006x over previous
import functools

import jax
import jax.numpy as jnp
from jax import lax
from jax.experimental import pallas as pl
from jax.experimental.pallas import tpu as pltpu
from jax.experimental.pallas import tpu_sc as plsc

_C = 128
_NW = 32


def _deg_partials(ei, ew, n_pad):
    e = ew.shape[0]
    nchunk = e // _C
    per_tile = n_pad // 16
    mesh = plsc.VectorSubcoreMesh(core_axis_name="c", subcore_axis_name="s")

    @functools.partial(
        pl.kernel,
        mesh=mesh,
        out_type=jax.ShapeDtypeStruct((2, n_pad), jnp.float32),
        scratch_types=[
            pltpu.VMEM((_C,), jnp.int32),
            pltpu.VMEM((_C,), jnp.float32),
            pltpu.VMEM((per_tile,), jnp.float32),
            pltpu.VMEM_SHARED((n_pad,), jnp.float32),
        ],
    )
    def k(ei_hbm, ew_hbm, out_hbm, col_v, ew_v, zb, deg_acc):
        cid = lax.axis_index("c")
        sid = lax.axis_index("s")
        wid = sid * 2 + cid

        def zero_body(i, _):
            zb[pl.ds(i * 16, 16)] = jnp.zeros((16,), jnp.float32)
            return 0

        lax.fori_loop(0, per_tile // 16, zero_body, 0)
        pltpu.sync_copy(zb, deg_acc.at[pl.ds(sid * per_tile, per_tile)])
        plsc.subcore_barrier()

        lo = wid * nchunk // _NW
        hi = (wid + 1) * nchunk // _NW

        def chunk(i, _):
            base = i * _C
            pltpu.sync_copy(ei_hbm.at[1, pl.ds(base, _C)], col_v)
            pltpu.sync_copy(ew_hbm.at[pl.ds(base, _C)], ew_v)
            pltpu.sync_copy(ew_v, deg_acc.at[col_v], add=True)
            return 0

        lax.fori_loop(lo, hi, chunk, 0)
        plsc.subcore_barrier()
        pltpu.sync_copy(
            deg_acc.at[pl.ds(sid * per_tile, per_tile)],
            out_hbm.at[cid, pl.ds(sid * per_tile, per_tile)],
        )

    return k(ei, ew)


def _prep(deg_partials, x, n, n_pad, d):
    def body(degp_ref, x_ref, dinv_ref, xs_ref):
        deg = degp_ref[0, :] + degp_ref[1, :] + 1.0
        dinv = lax.rsqrt(deg)
        dinv_ref[...] = dinv[:, None]
        xs_ref[...] = x_ref[...] * dinv[:n, None]

    return pl.pallas_call(
        body,
        out_shape=(
            jax.ShapeDtypeStruct((n_pad, 1), jnp.float32),
            jax.ShapeDtypeStruct((n, d), jnp.float32),
        ),
    )(deg_partials, x)


def _scatter_partials(xs, ei, ew, n_pad, d):
    e = ew.shape[0]
    nchunk = e // _C
    rows_per_tile = n_pad // 16
    zrows = _C
    mesh = plsc.VectorSubcoreMesh(core_axis_name="c", subcore_axis_name="s")

    @functools.partial(
        pl.kernel,
        mesh=mesh,
        out_type=jax.ShapeDtypeStruct((2, n_pad, d), jnp.float32),
        scratch_types=[
            pltpu.VMEM((_C,), jnp.int32),
            pltpu.VMEM((_C,), jnp.int32),
            pltpu.VMEM((_C,), jnp.float32),
            pltpu.VMEM((_C, d), jnp.float32),
            pltpu.VMEM_SHARED((n_pad, d), jnp.float32),
            pltpu.SemaphoreType.DMA,
        ],
    )
    def k(xs_hbm, ei_hbm, ew_hbm, out_hbm, row_v, col_v, ew_v, rows_v, p_acc, sem):
        cid = lax.axis_index("c")
        sid = lax.axis_index("s")
        wid = sid * 2 + cid

        def zero_body(i, _):
            r = i // 8
            kk = i % 8
            rows_v[r, pl.ds(kk * 16, 16)] = jnp.zeros((16,), jnp.float32)
            return 0

        lax.fori_loop(0, _C * 8, zero_body, 0)
        for j in range(rows_per_tile // zrows):
            pltpu.sync_copy(
                rows_v,
                p_acc.at[pl.ds(sid * rows_per_tile + j * zrows, zrows)],
            )
        plsc.subcore_barrier()

        lo = wid * nchunk // _NW
        hi = (wid + 1) * nchunk // _NW

        def chunk(i, _):
            base = i * _C
            pltpu.sync_copy(ei_hbm.at[0, pl.ds(base, _C)], row_v)
            pltpu.sync_copy(ei_hbm.at[1, pl.ds(base, _C)], col_v)
            pltpu.sync_copy(ew_hbm.at[pl.ds(base, _C)], ew_v)
            pltpu.async_copy(xs_hbm.at[row_v], rows_v, sem).wait()

            def scale(g, _):
                ew16 = ew_v[pl.ds(g * 16, 16)]
                for j in range(16):
                    w = ew16[j]
                    ee = g * 16 + j
                    for kk in range(8):
                        sl = rows_v[ee, pl.ds(kk * 16, 16)]
                        rows_v[ee, pl.ds(kk * 16, 16)] = sl * w
                return 0

            lax.fori_loop(0, _C // 16, scale, 0)
            pltpu.sync_copy(rows_v, p_acc.at[col_v], add=True)
            return 0

        lax.fori_loop(lo, hi, chunk, 0)
        plsc.subcore_barrier()
        pltpu.sync_copy(
            p_acc.at[pl.ds(sid * rows_per_tile, rows_per_tile)],
            out_hbm.at[cid, pl.ds(sid * rows_per_tile, rows_per_tile)],
        )

    return k(xs, ei, ew)


def _dense(pa, pb, xs, dinv2, mz, cz, mh, ch, hw, hb, n, d):
    blk = 1000

    def body(pa_ref, pb_ref, xs_ref, dinv_ref, mz_ref, cz_ref, mh_ref, ch_ref,
             hw_ref, hb_ref, out_ref, h_ref):
        p = dinv_ref[...] * (pa_ref[...] + pb_ref[...] + xs_ref[...])
        z = jax.nn.sigmoid(
            jnp.dot(p, mz_ref[...], preferred_element_type=jnp.float32,
                    precision=lax.Precision.HIGHEST) + cz_ref[...])
        ht = jnp.tanh(
            jnp.dot(p, mh_ref[...], preferred_element_type=jnp.float32,
                    precision=lax.Precision.HIGHEST) + ch_ref[...])
        h = (1.0 - z) * ht
        h_ref[...] = h
        out_ref[...] = jnp.dot(h, hw_ref[...], preferred_element_type=jnp.float32,
                               precision=lax.Precision.HIGHEST) + hb_ref[...]

    return pl.pallas_call(
        body,
        grid=(n // blk,),
        in_specs=[
            pl.BlockSpec((blk, d), lambda i: (i, 0)),
            pl.BlockSpec((blk, d), lambda i: (i, 0)),
            pl.BlockSpec((blk, d), lambda i: (i, 0)),
            pl.BlockSpec((blk, 1), lambda i: (i, 0)),
            pl.BlockSpec((d, d), lambda i: (0, 0)),
            pl.BlockSpec((1, d), lambda i: (0, 0)),
            pl.BlockSpec((d, d), lambda i: (0, 0)),
            pl.BlockSpec((1, d), lambda i: (0, 0)),
            pl.BlockSpec((d, 1), lambda i: (0, 0)),
            pl.BlockSpec((1, 1), lambda i: (0, 0)),
        ],
        out_specs=(
            pl.BlockSpec((blk, 1), lambda i: (i, 0)),
            pl.BlockSpec((blk, d), lambda i: (i, 0)),
        ),
        out_shape=(
            jax.ShapeDtypeStruct((n, 1), jnp.float32),
            jax.ShapeDtypeStruct((n, d), jnp.float32),
        ),
    )(pa, pb, xs, dinv2, mz, cz, mh, ch, hw, hb)


def kernel(x, ei, ew, Wz, bz, Lz_w, Lz_b, Wr, br, Lr_w, Lr_b, Wh, bh, Lh_w,
           Lh_b, head_w, head_b):
    n, d = x.shape
    hid = Wz.shape[1]
    n_pad = ((n + 255) // 256) * 256

    degp = _deg_partials(ei, ew, n_pad)
    dinv2, xs = _prep(degp, x, n, n_pad, d)
    pp = _scatter_partials(xs, ei, ew, n_pad, d)

    mz = Wz @ Lz_w[:hid]
    cz = (bz @ Lz_w[:hid] + Lz_b)[None, :]
    mh = Wh @ Lh_w[:hid]
    ch = (bh @ Lh_w[:hid] + Lh_b)[None, :]

    out, h_new = _dense(pp[0, :n], pp[1, :n], xs, dinv2[:n], mz, cz, mh, ch,
                        head_w, head_b[None, :], n, d)
    return (out, h_new)
